# Initial kernel scaffold; baseline (speedup 1.0000x reference)
import jax
import jax.numpy as jnp
from jax.experimental import pallas as pl


def kernel(input_ids, identity_mask, table, W1, b1, W2, b2):
    raise NotImplementedError("write your pallas kernel here")


# speedup vs baseline: 2.5730x; 2.5730x over previous
import functools

import jax
import jax.numpy as jnp
from jax import lax
from jax.experimental import pallas as pl
from jax.experimental.pallas import tpu as pltpu
from jax.experimental.pallas import tpu_sc as plsc

B, L = 16384, 50
V, D = 1000000, 64
H, C = 64, 2

NW = 32
SEQ_PER_W = B // NW
S_CHUNK = 8
CT = S_CHUNK * L
NCH = SEQ_PER_W // S_CHUNK
GATHER_SPLITS = [(0, 128), (128, 128), (256, 128), (384, 16)]
LANES = 16


def _sc_pool_body(ids_hbm, mask_hbm, table_hbm, out_hbm,
                  idx0, idx1, m0, m1, rows0, rows1, wbuf, outv, sem0, sem1):
    c = lax.axis_index("c")
    s = lax.axis_index("s")
    wid = s * 2 + c
    tok_base = wid * (SEQ_PER_W * L)
    seq_base = wid * SEQ_PER_W

    idx = [idx0, idx1]
    msk = [m0, m1]
    rows = [rows0, rows1]
    sems = [sem0, sem1]

    def load_and_fire(k, b):
        tb = tok_base + k * CT
        pltpu.sync_copy(ids_hbm.at[pl.ds(tb, CT)], idx[b])
        pltpu.sync_copy(mask_hbm.at[pl.ds(tb, CT)], msk[b])
        for (o, n) in GATHER_SPLITS:
            pltpu.async_copy(table_hbm.at[idx[b].at[pl.ds(o, n)]],
                             rows[b].at[pl.ds(o, n)], sems[b])

    def wait_gathers(b):
        for (o, n) in GATHER_SPLITS:
            pltpu.make_async_copy(table_hbm.at[idx[b].at[pl.ds(o, n)]],
                                  rows[b].at[pl.ds(o, n)], sems[b]).wait()

    def compute_weights(b):
        for t in range(CT // LANES):
            ids16 = idx[b][pl.ds(t * LANES, LANES)]
            mm16 = msk[b][pl.ds(t * LANES, LANES)]
            keep = (ids16 != 0) & (mm16 != 0)
            wbuf[pl.ds(t * LANES, LANES)] = jnp.where(keep, 1.0, 0.0)

    def accumulate(k, b):
        def seq_body(si, carry):
            tb = si * L
            wch = [wbuf[pl.ds(tb, LANES)],
                   wbuf[pl.ds(tb + 16, LANES)],
                   wbuf[pl.ds(tb + 32, LANES)],
                   wbuf[pl.ds(tb + 34, LANES)]]
            acc = [jnp.zeros((LANES,), jnp.float32) for _ in range(4)]
            for l in range(L):
                w = wch[l // 16][l % 16] if l < 48 else wch[3][l - 34]
                for j in range(4):
                    acc[j] = acc[j] + w * rows[b][tb + l, pl.ds(j * LANES, LANES)]
            for j in range(4):
                outv[si, pl.ds(j * LANES, LANES)] = acc[j]
            return carry
        lax.fori_loop(0, S_CHUNK, seq_body, 0)
        pltpu.sync_copy(outv, out_hbm.at[pl.ds(seq_base + k * S_CHUNK, S_CHUNK)])

    load_and_fire(0, 0)

    def pair_body(kk, carry):
        for b in (0, 1):
            k = kk * 2 + b

            @pl.when(k + 1 < NCH)
            def _():
                load_and_fire(k + 1, 1 - b)

            compute_weights(b)
            wait_gathers(b)
            accumulate(k, b)
        return carry

    lax.fori_loop(0, NCH // 2, pair_body, 0)


@functools.partial(jax.jit, static_argnames=())
def _sc_pool(ids_flat, mask_flat, table):
    kfn = pl.kernel(
        _sc_pool_body,
        out_type=jax.ShapeDtypeStruct((B, D), jnp.float32),
        mesh=plsc.VectorSubcoreMesh(core_axis_name="c", subcore_axis_name="s"),
        compiler_params=pltpu.CompilerParams(use_tc_tiling_on_sc=False),
        scratch_types=[
            pltpu.VMEM((CT,), jnp.int32),
            pltpu.VMEM((CT,), jnp.int32),
            pltpu.VMEM((CT,), jnp.int32),
            pltpu.VMEM((CT,), jnp.int32),
            pltpu.VMEM((CT, D), jnp.float32),
            pltpu.VMEM((CT, D), jnp.float32),
            pltpu.VMEM((CT,), jnp.float32),
            pltpu.VMEM((S_CHUNK, D), jnp.float32),
            pltpu.SemaphoreType.DMA,
            pltpu.SemaphoreType.DMA,
        ],
    )
    return kfn(ids_flat, mask_flat, table)


BS = 1024


def _mlp_body(sum_ref, mask_ref, w1t_ref, b1_ref, w2p_ref, b2p_ref, out_ref):
    cnt = jnp.sum(mask_ref[...].astype(jnp.float32), axis=1, keepdims=True)
    pooled = sum_ref[...] / (cnt + 1e-9)
    h = jnp.dot(pooled, w1t_ref[...], preferred_element_type=jnp.float32)
    h = jnp.maximum(h + b1_ref[...], 0.0)
    out_ref[...] = (jnp.dot(h, w2p_ref[...], preferred_element_type=jnp.float32)
                    + b2p_ref[...])


def _mlp(pooled_sums, identity_mask, W1, b1, W2, b2):
    w1t = W1.T
    b1r = b1.reshape(1, H)
    w2p = jnp.zeros((H, 128), jnp.float32).at[:, :C].set(W2.T)
    b2p = jnp.zeros((1, 128), jnp.float32).at[0, :C].set(b2)
    out_pad = pl.pallas_call(
        _mlp_body,
        grid=(B // BS,),
        in_specs=[
            pl.BlockSpec((BS, D), lambda i: (i, 0)),
            pl.BlockSpec((BS, L), lambda i: (i, 0)),
            pl.BlockSpec((D, H), lambda i: (0, 0)),
            pl.BlockSpec((1, H), lambda i: (0, 0)),
            pl.BlockSpec((H, 128), lambda i: (0, 0)),
            pl.BlockSpec((1, 128), lambda i: (0, 0)),
        ],
        out_specs=pl.BlockSpec((BS, 128), lambda i: (i, 0)),
        out_shape=jax.ShapeDtypeStruct((B, 128), jnp.float32),
    )(pooled_sums, identity_mask, w1t, b1r, w2p, b2p)
    return out_pad[:, :C]


def kernel(input_ids, identity_mask, table, W1, b1, W2, b2):
    ids_flat = input_ids.reshape(B * L)
    mask_flat = identity_mask.reshape(B * L)
    pooled_sums = _sc_pool(ids_flat, mask_flat, table)
    return _mlp(pooled_sums, identity_mask, W1, b1, W2, b2)
